# baseline (speedup 1.0000x reference)
import functools

import jax
import jax.numpy as jnp
from jax import lax
from jax.experimental import pallas as pl
from jax.experimental.pallas import tpu as pltpu
from jax.experimental.pallas import tpu_sc as plsc

FACTOR = 32
BATCH = 16384
LANES = 16
CHUNK = 128

_info = plsc.get_sparse_core_info()
NC, NS = _info.num_cores, _info.num_subcores
NW = NC * NS
B_PER_W = BATCH // NW
NCHUNK = B_PER_W // CHUNK
NGROUP = B_PER_W // LANES


def _gmf_body(user_hbm, item_hbm, tab_u, tab_i, w_hbm, b_hbm, out_hbm,
              idx_u, idx_i, rows_u, rows_i, w_v, b_v, out_v, sem):
    wid = lax.axis_index("s") * NC + lax.axis_index("c")
    base = wid * B_PER_W

    pltpu.sync_copy(user_hbm.at[wid], idx_u)
    pltpu.sync_copy(item_hbm.at[wid], idx_i)
    pltpu.sync_copy(w_hbm, w_v)
    pltpu.sync_copy(b_hbm, b_v)

    copies = []
    for j in range(NCHUNK):
        dst = pl.ds(j * CHUNK, CHUNK)
        copies.append(pltpu.async_copy(tab_u.at[idx_u.at[j]], rows_u.at[dst], sem))
        copies.append(pltpu.async_copy(tab_i.at[idx_i.at[j]], rows_i.at[dst], sem))
    for c in copies:
        c.wait()

    w0 = w_v[pl.ds(0, LANES)]
    w1 = w_v[pl.ds(LANES, LANES)]
    bias = b_v[...]
    lidx = lax.iota(jnp.int32, LANES)

    def group(g, carry):
        base_r = g * LANES
        acc = bias
        for k in range(LANES):
            r = base_r + k
            u0 = rows_u[r, pl.ds(0, LANES)]
            u1 = rows_u[r, pl.ds(LANES, LANES)]
            v0 = rows_i[r, pl.ds(0, LANES)]
            v1 = rows_i[r, pl.ds(LANES, LANES)]
            t = u0 * v0 * w0 + u1 * v1 * w1
            acc = jnp.where(lidx == k, acc + jnp.sum(t), acc)
        out_v[pl.ds(base_r, LANES)] = acc
        return carry

    lax.fori_loop(0, NGROUP, group, 0)
    pltpu.sync_copy(out_v, out_hbm.at[pl.ds(base, B_PER_W)])


_gmf = functools.partial(
    pl.kernel,
    mesh=plsc.VectorSubcoreMesh(core_axis_name="c", subcore_axis_name="s"),
    out_type=jax.ShapeDtypeStruct((BATCH,), jnp.float32),
    compiler_params=pltpu.CompilerParams(
        needs_layout_passes=False, use_tc_tiling_on_sc=False),
    scratch_types=[
        pltpu.VMEM((NCHUNK, CHUNK), jnp.int32),
        pltpu.VMEM((NCHUNK, CHUNK), jnp.int32),
        pltpu.VMEM((B_PER_W, FACTOR), jnp.float32),
        pltpu.VMEM((B_PER_W, FACTOR), jnp.float32),
        pltpu.VMEM((FACTOR,), jnp.float32),
        pltpu.VMEM((LANES,), jnp.float32),
        pltpu.VMEM((B_PER_W,), jnp.float32),
        pltpu.SemaphoreType.DMA,
    ],
)(_gmf_body)


def kernel(user, item, embed_user_GMF, embed_item_GMF, predict_W, predict_b):
    user_r = user.astype(jnp.int32).reshape(NW, NCHUNK, CHUNK)
    item_r = item.astype(jnp.int32).reshape(NW, NCHUNK, CHUNK)
    w_b = predict_W.reshape(FACTOR)
    b_b = jnp.broadcast_to(predict_b.reshape(1), (LANES,))
    return _gmf(user_r, item_r, embed_user_GMF, embed_item_GMF, w_b, b_b)
